# baseline (speedup 1.0000x reference)
import functools

import jax
import jax.numpy as jnp
from jax import lax
from jax.experimental import pallas as pl
from jax.experimental.pallas import tpu as pltpu
from jax.experimental.pallas import tpu_sc as plsc


TOPK = 8
NUM_EXPERTS = 64
CHUNK = 512
NBUF = 4
LOOKAHEAD = 3
N_ROWS = 8192
LANES = 16


def _probs_kernel(x_ref, w_ref, p_out_ref, buf, sems, wt_scr):
    i = pl.program_id(0)
    n = pl.num_programs(0)

    def start(j):
        slot = lax.rem(j, NBUF)
        pltpu.make_async_copy(
            x_ref.at[pl.ds(j * CHUNK, CHUNK), :],
            buf.at[slot],
            sems.at[slot],
        ).start()

    @pl.when(i == 0)
    def _prologue():
        for j in range(LOOKAHEAD):
            start(jnp.int32(j))
        wt_scr[...] = w_ref[...].T

    @pl.when(i + LOOKAHEAD < n)
    def _prefetch():
        start(i + LOOKAHEAD)

    slot = lax.rem(i, NBUF)
    pltpu.make_async_copy(
        x_ref.at[pl.ds(i * CHUNK, CHUNK), :],
        buf.at[slot],
        sems.at[slot],
    ).wait()

    x = buf[slot]
    scores = jnp.dot(x, wt_scr[...], preferred_element_type=jnp.float32)
    st = scores.T
    m = jnp.max(st, axis=0, keepdims=True)
    e = jnp.exp(st - m)
    s = jnp.sum(e, axis=0, keepdims=True)
    p_out_ref[...] = e / s


def _probs_call(x, weight):
    n_rows = x.shape[0]
    dim = x.shape[1]
    grid = (n_rows // CHUNK,)
    return pl.pallas_call(
        _probs_kernel,
        grid=grid,
        in_specs=[
            pl.BlockSpec(memory_space=pl.ANY),
            pl.BlockSpec((NUM_EXPERTS, dim), lambda i: (0, 0)),
        ],
        out_specs=pl.BlockSpec((NUM_EXPERTS, CHUNK), lambda i: (0, i)),
        out_shape=jax.ShapeDtypeStruct((NUM_EXPERTS, n_rows), jnp.float32),
        scratch_shapes=[
            pltpu.VMEM((NBUF, CHUNK, dim), jnp.float32),
            pltpu.SemaphoreType.DMA((NBUF,)),
            pltpu.VMEM((dim, NUM_EXPERTS), jnp.float32),
        ],
    )(x, weight)


def _sc_topk_kernel(p_ref, w_out_ref, i_out_ref, p_vmem, w_vmem, i_vmem):
    info = plsc.get_sparse_core_info()
    nc, ns = info.num_cores, info.num_subcores
    nw = nc * ns
    tpw = N_ROWS // nw
    wid = lax.axis_index("s") * nc + lax.axis_index("c")
    base = wid * tpw

    pltpu.sync_copy(p_ref.at[:, pl.ds(base * 1, tpw)], p_vmem)

    lane_iota = lax.iota(jnp.int32, LANES)

    def group_body(g, carry):
        col0 = g * LANES
        mp = jnp.full((LANES,), 2.0, jnp.float32)
        mip = jnp.full((LANES,), -1, jnp.int32)
        for r in range(TOPK):
            m = jnp.full((LANES,), -1.0, jnp.float32)
            mi = jnp.zeros((LANES,), jnp.int32)
            for ex in range(NUM_EXPERTS):
                v = p_vmem[ex, pl.ds(col0, LANES)]
                valid = (v < mp) | ((v == mp) & (ex > mip))
                c = valid & (v > m)
                m = jnp.where(c, v, m)
                mi = jnp.where(c, ex, mi)
            w_vmem[r, pl.ds(col0, LANES)] = m
            i_vmem[r, pl.ds(col0, LANES)] = mi
            mp = m
            mip = mi
        return carry

    lax.fori_loop(0, tpw // LANES, group_body, jnp.int32(0))

    pltpu.sync_copy(w_vmem, w_out_ref.at[:, pl.ds(base, tpw)])
    pltpu.sync_copy(i_vmem, i_out_ref.at[:, pl.ds(base, tpw)])


def _sc_topk_call(probs_t):
    n_rows = probs_t.shape[1]
    info = plsc.get_sparse_core_info()
    tpw = n_rows // (info.num_cores * info.num_subcores)
    mesh = plsc.VectorSubcoreMesh(core_axis_name="c", subcore_axis_name="s")
    k = functools.partial(
        pl.kernel,
        mesh=mesh,
        out_type=[
            jax.ShapeDtypeStruct((TOPK, n_rows), jnp.float32),
            jax.ShapeDtypeStruct((TOPK, n_rows), jnp.int32),
        ],
        scratch_types=[
            pltpu.VMEM((NUM_EXPERTS, tpw), jnp.float32),
            pltpu.VMEM((TOPK, tpw), jnp.float32),
            pltpu.VMEM((TOPK, tpw), jnp.int32),
        ],
    )(_sc_topk_kernel)
    return k(probs_t)


def _tr_kernel(wt_ref, it_ref, w_out_ref, i_out_ref):
    w_out_ref[...] = wt_ref[...].T
    i_out_ref[...] = it_ref[...].T


def _tr_call(w_t, i_t):
    n_rows = w_t.shape[1]
    return pl.pallas_call(
        _tr_kernel,
        out_shape=[
            jax.ShapeDtypeStruct((n_rows, TOPK), jnp.float32),
            jax.ShapeDtypeStruct((n_rows, TOPK), jnp.int32),
        ],
    )(w_t, i_t)


@functools.partial(jax.jit, static_argnames=())
def kernel(x, weight):
    probs_t = _probs_call(x, weight)
    w_t, i_t = _sc_topk_call(probs_t)
    weights_out, indices_out = _tr_call(w_t, i_t)
    return weights_out, indices_out


# speedup vs baseline: 2.7682x; 2.7682x over previous
import functools

import jax
import jax.numpy as jnp
from jax.experimental import pallas as pl
from jax.experimental.pallas import tpu as pltpu


TOPK = 8
NUM_EXPERTS = 64
CHUNK = 512
NBUF = 6
LOOKAHEAD = 5


def _topk_block(scores):
    st = scores.T
    m = jnp.max(st, axis=0, keepdims=True)
    e = jnp.exp(st - m)
    s = jnp.sum(e, axis=0, keepdims=True)
    p = e / s
    iota = jax.lax.broadcasted_iota(jnp.int32, p.shape, 0)
    vals = []
    idxs = []
    for _ in range(TOPK):
        mk = jnp.max(p, axis=0, keepdims=True)
        ik = jnp.min(jnp.where(p == mk, iota, NUM_EXPERTS), axis=0,
                     keepdims=True)
        vals.append(mk)
        idxs.append(ik)
        p = jnp.where(iota == ik, -1.0, p)
    return (jnp.concatenate(vals, axis=0).T,
            jnp.concatenate(idxs, axis=0).T)


def _router_kernel(x_ref, w_ref, w_out_ref, i_out_ref, buf, sems, wt_scr):
    i = pl.program_id(0)
    n = pl.num_programs(0)

    def start(j):
        slot = jax.lax.rem(j, NBUF)
        pltpu.make_async_copy(
            x_ref.at[pl.ds(j * CHUNK, CHUNK), :],
            buf.at[slot],
            sems.at[slot],
        ).start()

    @pl.when(i == 0)
    def _prologue():
        for j in range(LOOKAHEAD):
            start(jnp.int32(j))
        wt_scr[...] = w_ref[...].T

    @pl.when(i + LOOKAHEAD < n)
    def _prefetch():
        start(i + LOOKAHEAD)

    slot = jax.lax.rem(i, NBUF)
    pltpu.make_async_copy(
        x_ref.at[pl.ds(i * CHUNK, CHUNK), :],
        buf.at[slot],
        sems.at[slot],
    ).wait()

    x = buf[slot]
    scores = jnp.dot(x, wt_scr[...], preferred_element_type=jnp.float32)
    w_vals, w_idxs = _topk_block(scores)
    w_out_ref[...] = w_vals
    i_out_ref[...] = w_idxs


@functools.partial(jax.jit, static_argnames=())
def kernel(x, weight):
    n_rows = x.shape[0]
    dim = x.shape[1]
    grid = (n_rows // CHUNK,)
    weights_out, indices_out = pl.pallas_call(
        _router_kernel,
        grid=grid,
        in_specs=[
            pl.BlockSpec(memory_space=pl.ANY),
            pl.BlockSpec((NUM_EXPERTS, dim), lambda i: (0, 0)),
        ],
        out_specs=[
            pl.BlockSpec((CHUNK, TOPK), lambda i: (i, 0)),
            pl.BlockSpec((CHUNK, TOPK), lambda i: (i, 0)),
        ],
        out_shape=[
            jax.ShapeDtypeStruct((n_rows, TOPK), jnp.float32),
            jax.ShapeDtypeStruct((n_rows, TOPK), jnp.int32),
        ],
        scratch_shapes=[
            pltpu.VMEM((NBUF, CHUNK, dim), jnp.float32),
            pltpu.SemaphoreType.DMA((NBUF,)),
            pltpu.VMEM((dim, NUM_EXPERTS), jnp.float32),
        ],
    )(x, weight)
    return weights_out, indices_out
